# baseline (speedup 1.0000x reference)
import functools

import jax
import jax.numpy as jnp
from jax import lax
from jax.experimental import pallas as pl
from jax.experimental.pallas import tpu as pltpu
from jax.experimental.pallas import tpu_sc as plsc

B, NJ, COL = 4096, 14, 14
CELLS = COL * COL
NP = B * NJ
HN = NP * CELLS

TC_ROWS, TC_COLS = 256, 43904
TC_BLK = 16


def _sumsq_body(h_ref, out_ref):
    @pl.when(pl.program_id(0) == 0)
    def _():
        out_ref[0, 0] = 0.0
    x = h_ref[...]
    out_ref[0, 0] += jnp.sum(x * x)


def _sumsq(h2d):
    return pl.pallas_call(
        _sumsq_body,
        grid=(TC_ROWS // TC_BLK,),
        in_specs=[pl.BlockSpec((TC_BLK, TC_COLS), lambda i: (i, 0))],
        out_specs=pl.BlockSpec((1, 1), lambda i: (0, 0),
                               memory_space=pltpu.SMEM),
        out_shape=jax.ShapeDtypeStruct((1, 1), jnp.float32),
    )(h2d)


NC, NS, L = 2, 16, 16
NW = NC * NS
PPW = NP // NW
GROWS = PPW // 128

_sc_mesh = plsc.VectorSubcoreMesh(core_axis_name="c", subcore_axis_name="s")


@functools.partial(
    pl.kernel,
    mesh=_sc_mesh,
    out_type=jax.ShapeDtypeStruct((NW, L), jnp.float32),
    scratch_types=[
        pltpu.VMEM((PPW,), jnp.float32),
        pltpu.VMEM((PPW,), jnp.float32),
        pltpu.VMEM((PPW,), jnp.int32),
        pltpu.VMEM((GROWS, 128), jnp.int32),
        pltpu.VMEM((GROWS, 128), jnp.float32),
        pltpu.VMEM((L,), jnp.float32),
        pltpu.SemaphoreType.DMA,
    ],
)
def _corr_kernel(tx_hbm, ty_hbm, v_hbm, h_hbm, out_hbm,
                 tx_v, ty_v, v_v, idx_v, g_v, acc_v, sem):
    wid = lax.axis_index("s") * NC + lax.axis_index("c")
    base = wid * PPW
    pltpu.sync_copy(tx_hbm.at[pl.ds(base, PPW)], tx_v)
    pltpu.sync_copy(ty_hbm.at[pl.ds(base, PPW)], ty_v)
    pltpu.sync_copy(v_hbm.at[pl.ds(base, PPW)], v_v)

    lane = lax.iota(jnp.int32, L)

    def idx_chunk(i, _):
        tx = tx_v[pl.ds(i * L, L)]
        ty = ty_v[pl.ds(i * L, L)]
        xi = jnp.clip((tx * COL).astype(jnp.int32), 0, COL - 1)
        yi = jnp.clip((ty * COL).astype(jnp.int32), 0, COL - 1)
        p = base + i * L + lane
        row = i // 8
        col = (i % 8) * L
        idx_v[row, pl.ds(col, L)] = p * CELLS + xi * COL + yi
        return 0

    lax.fori_loop(0, PPW // L, idx_chunk, 0)

    copies = [
        pltpu.async_copy(h_hbm.at[idx_v.at[j]], g_v.at[j], sem)
        for j in range(GROWS)
    ]
    for cp in copies:
        cp.wait()

    def acc_chunk(i, acc):
        g = g_v[i // 8, pl.ds((i % 8) * L, L)]
        vis = v_v[pl.ds(i * L, L)]
        return acc + jnp.where(vis == 1, 1.0 - 2.0 * g, 0.0)

    acc = lax.fori_loop(0, PPW // L, acc_chunk, jnp.zeros((L,), jnp.float32))
    acc_v[...] = acc
    pltpu.sync_copy(acc_v, out_hbm.at[wid])


def kernel(o, h, t, v):
    h2d = h.reshape(TC_ROWS, TC_COLS)
    hflat = h.reshape(HN)
    tx = t[:, :, 0].reshape(NP)
    ty = t[:, :, 1].reshape(NP)
    vf = v.reshape(NP)
    ssq = _sumsq(h2d)[0, 0]
    parts = _corr_kernel(tx, ty, vf, hflat)
    return (ssq + jnp.sum(parts)) / jnp.float32(HN / 2.0)


# speedup vs baseline: 22.3306x; 22.3306x over previous
import jax
import jax.numpy as jnp
from jax import lax
from jax.experimental import pallas as pl
from jax.experimental.pallas import tpu as pltpu

B, NJ, COL = 4096, 14, 14
CELLS = COL * COL
ROWS = NJ * COL
BX = 7
GRID = ROWS // BX
JPB = COL // BX


def _mse_body(h_ref, t_ref, v_ref, out_ref):
    i = pl.program_id(0)

    @pl.when(i == 0)
    def _():
        out_ref[0, 0] = 0.0

    tx = t_ref[0, 0]
    ty = t_ref[0, 1]
    vis = v_ref[0, 0]
    xi = jnp.clip((tx * COL).astype(jnp.int32), 0, COL - 1)
    yi = jnp.clip((ty * COL).astype(jnp.int32), 0, COL - 1)
    cell = jnp.where(vis == 1, xi * COL + yi, -1)

    x = h_ref[...]
    cx = lax.broadcasted_iota(jnp.int32, (BX, COL, B), 0) + (i % JPB) * BX
    cy = lax.broadcasted_iota(jnp.int32, (BX, COL, B), 1)
    m = (cx * COL + cy == cell[None, None, :]).astype(jnp.float32)
    d = x - m
    out_ref[0, 0] += jnp.sum(d * d)


def _mse(h3, tT, vT):
    return pl.pallas_call(
        _mse_body,
        grid=(GRID,),
        in_specs=[
            pl.BlockSpec((BX, COL, B), lambda i: (i, 0, 0)),
            pl.BlockSpec((1, 2, B), lambda i: (i // JPB, 0, 0)),
            pl.BlockSpec((1, 1, B), lambda i: (i // JPB, 0, 0)),
        ],
        out_specs=pl.BlockSpec((1, 1), lambda i: (0, 0),
                               memory_space=pltpu.SMEM),
        out_shape=jax.ShapeDtypeStruct((1, 1), jnp.float32),
    )(h3, tT, vT)


def kernel(o, h, t, v):
    h3 = h.transpose(1, 2, 3, 0).reshape(ROWS, COL, B)
    tT = t.transpose(1, 2, 0)
    vT = v.transpose(1, 2, 0)
    total = _mse(h3, tT, vT)[0, 0]
    return total / jnp.float32(B * NJ * CELLS / 2.0)


# speedup vs baseline: 29.2653x; 1.3105x over previous
import jax
import jax.numpy as jnp
from jax import lax
from jax.experimental import pallas as pl
from jax.experimental.pallas import tpu as pltpu

B, NJ, COL = 4096, 14, 14
CELLS = COL * COL
JB = 1
GRID = NJ // JB


def _mse_body(h_ref, t_ref, v_ref, out_ref):
    i = pl.program_id(0)

    @pl.when(i == 0)
    def _():
        out_ref[0, 0] = 0.0

    tx = t_ref[:, 0]
    ty = t_ref[:, 1]
    vis = v_ref[:, 0]
    xi = jnp.clip((tx * COL).astype(jnp.int32), 0, COL - 1)
    yi = jnp.clip((ty * COL).astype(jnp.int32), 0, COL - 1)
    cell = jnp.where(vis == 1, xi * COL + yi, -1)

    x = h_ref[...]
    cx = lax.broadcasted_iota(jnp.int32, (JB, COL, COL, B), 1)
    cy = lax.broadcasted_iota(jnp.int32, (JB, COL, COL, B), 2)
    m = (cx * COL + cy == cell[:, None, None, :]).astype(jnp.float32)
    d = x - m
    out_ref[0, 0] += jnp.sum(d * d)


def _mse(h4, tT, vT):
    return pl.pallas_call(
        _mse_body,
        grid=(GRID,),
        in_specs=[
            pl.BlockSpec((JB, COL, COL, B), lambda i: (i, 0, 0, 0)),
            pl.BlockSpec((JB, 2, B), lambda i: (i, 0, 0)),
            pl.BlockSpec((JB, 1, B), lambda i: (i, 0, 0)),
        ],
        out_specs=pl.BlockSpec((1, 1), lambda i: (0, 0),
                               memory_space=pltpu.SMEM),
        out_shape=jax.ShapeDtypeStruct((1, 1), jnp.float32),
    )(h4, tT, vT)


def kernel(o, h, t, v):
    h4 = h.transpose(1, 2, 3, 0)
    tT = t.transpose(1, 2, 0)
    vT = v.transpose(1, 2, 0)
    total = _mse(h4, tT, vT)[0, 0]
    return total / jnp.float32(B * NJ * CELLS / 2.0)
